# Initial kernel scaffold; baseline (speedup 1.0000x reference)
import jax
import jax.numpy as jnp
from jax.experimental import pallas as pl


def kernel(student_id, engagement_id, student_table, engagement_table):
    raise NotImplementedError("write your pallas kernel here")


# speedup vs baseline: 1.2579x; 1.2579x over previous
import functools

import jax
import jax.numpy as jnp
from jax import lax
from jax.experimental import pallas as pl
from jax.experimental.pallas import tpu as pltpu
from jax.experimental.pallas import tpu_sc as plsc

_NUM_BINS = 10000
_EMBED_DIM = 64
_BATCH = 16384

_info = plsc.get_sparse_core_info()
_NC, _NS, _L = _info.num_cores, _info.num_subcores, _info.num_lanes
_NW = _NC * _NS
_BPW = _BATCH // _NW
_CHUNK = 128
_KCH = _BPW // _CHUNK


def _sc_body(sid_hbm, eid_hbm, stab_hbm, etab_hbm, out_hbm,
             ids_v, sidx, eidx, oidx_s, oidx_e, s_rows, e_rows, sem):
    wid = lax.axis_index("s") * _NC + lax.axis_index("c")
    base = wid * _BPW

    pltpu.sync_copy(sid_hbm.at[pl.ds(base, _BPW)], ids_v.at[0])
    pltpu.sync_copy(eid_hbm.at[pl.ds(base, _BPW)], ids_v.at[1])

    lane = lax.iota(jnp.int32, _L)
    for j in range(_KCH):
        for t in range(_CHUNK // _L):
            col = t * _L
            i0 = j * _CHUNK + col
            s = ids_v[0, pl.ds(i0, _L)]
            e = ids_v[1, pl.ds(i0, _L)]
            sidx[j, pl.ds(col, _L)] = s % _NUM_BINS
            eidx[j, pl.ds(col, _L)] = e % _NUM_BINS
            row = (base + i0) + lane
            oidx_s[j, pl.ds(col, _L)] = row * 2
            oidx_e[j, pl.ds(col, _L)] = row * 2 + 1

    descs = []
    for j in range(_KCH):
        dst = pl.ds(j * _CHUNK, _CHUNK)
        descs.append(pltpu.async_copy(stab_hbm.at[sidx.at[j]], s_rows.at[dst], sem))
        descs.append(pltpu.async_copy(etab_hbm.at[eidx.at[j]], e_rows.at[dst], sem))
    for d in descs:
        d.wait()

    descs = []
    for j in range(_KCH):
        src = pl.ds(j * _CHUNK, _CHUNK)
        descs.append(pltpu.async_copy(s_rows.at[src], out_hbm.at[oidx_s.at[j]], sem))
        descs.append(pltpu.async_copy(e_rows.at[src], out_hbm.at[oidx_e.at[j]], sem))
    for d in descs:
        d.wait()


_sc_call = functools.partial(
    pl.kernel,
    mesh=plsc.VectorSubcoreMesh(core_axis_name="c", subcore_axis_name="s"),
    out_type=jax.ShapeDtypeStruct((2 * _BATCH, _EMBED_DIM), jnp.float32),
    scratch_types=[
        pltpu.VMEM((2, _BPW), jnp.int32),
        pltpu.VMEM((_KCH, _CHUNK), jnp.int32),
        pltpu.VMEM((_KCH, _CHUNK), jnp.int32),
        pltpu.VMEM((_KCH, _CHUNK), jnp.int32),
        pltpu.VMEM((_KCH, _CHUNK), jnp.int32),
        pltpu.VMEM((_BPW, _EMBED_DIM), jnp.float32),
        pltpu.VMEM((_BPW, _EMBED_DIM), jnp.float32),
        pltpu.SemaphoreType.DMA,
    ],
    compiler_params=pltpu.CompilerParams(use_tc_tiling_on_sc=False),
)(_sc_body)


def kernel(student_id, engagement_id, student_table, engagement_table):
    out2 = _sc_call(student_id.astype(jnp.int32), engagement_id.astype(jnp.int32),
                    student_table, engagement_table)
    return out2.reshape(_BATCH, 2 * _EMBED_DIM)


# speedup vs baseline: 1.2624x; 1.0036x over previous
import functools

import jax
import jax.numpy as jnp
from jax import lax
from jax.experimental import pallas as pl
from jax.experimental.pallas import tpu as pltpu
from jax.experimental.pallas import tpu_sc as plsc

_NUM_BINS = 10000
_EMBED_DIM = 64
_BATCH = 16384

_info = plsc.get_sparse_core_info()
_NC, _NS, _L = _info.num_cores, _info.num_subcores, _info.num_lanes
_NW = _NC * _NS
_BPW = _BATCH // _NW
_CHUNK = 128
_KCH = _BPW // _CHUNK


def _sc_body(sid_hbm, eid_hbm, stab_hbm, etab_hbm, out_hbm,
             ids_v, sidx, eidx, comb, gsems, osem):
    wid = lax.axis_index("s") * _NC + lax.axis_index("c")
    base = wid * _BPW

    pltpu.sync_copy(sid_hbm.at[pl.ds(base, _BPW)], ids_v.at[0])
    pltpu.sync_copy(eid_hbm.at[pl.ds(base, _BPW)], ids_v.at[1])

    for j in range(_KCH):
        for t in range(_CHUNK // _L):
            col = t * _L
            i0 = j * _CHUNK + col
            sidx[j, pl.ds(col, _L)] = ids_v[0, pl.ds(i0, _L)] % _NUM_BINS
            eidx[j, pl.ds(col, _L)] = ids_v[1, pl.ds(i0, _L)] % _NUM_BINS

    gd = []
    for j in range(_KCH):
        rows = pl.ds(j * _CHUNK, _CHUNK)
        gd.append(pltpu.async_copy(stab_hbm.at[sidx.at[j]], comb.at[0, rows], gsems.at[j]))
        gd.append(pltpu.async_copy(etab_hbm.at[eidx.at[j]], comb.at[1, rows], gsems.at[j]))
    od = []
    for j in range(_KCH):
        gd[2 * j].wait()
        gd[2 * j + 1].wait()
        rows = pl.ds(j * _CHUNK, _CHUNK)
        orows = pl.ds(base + j * _CHUNK, _CHUNK)
        od.append(pltpu.async_copy(comb.at[0, rows],
                                   out_hbm.at[orows, pl.ds(0, _EMBED_DIM)], osem))
        od.append(pltpu.async_copy(comb.at[1, rows],
                                   out_hbm.at[orows, pl.ds(_EMBED_DIM, _EMBED_DIM)], osem))
    for d in od:
        d.wait()


_sc_call = functools.partial(
    pl.kernel,
    mesh=plsc.VectorSubcoreMesh(core_axis_name="c", subcore_axis_name="s"),
    out_type=jax.ShapeDtypeStruct((_BATCH, 2 * _EMBED_DIM), jnp.float32),
    scratch_types=[
        pltpu.VMEM((2, _BPW), jnp.int32),
        pltpu.VMEM((_KCH, _CHUNK), jnp.int32),
        pltpu.VMEM((_KCH, _CHUNK), jnp.int32),
        pltpu.VMEM((2, _BPW, _EMBED_DIM), jnp.float32),
        pltpu.SemaphoreType.DMA((_KCH,)),
        pltpu.SemaphoreType.DMA,
    ],
    compiler_params=pltpu.CompilerParams(use_tc_tiling_on_sc=False),
)(_sc_body)


def kernel(student_id, engagement_id, student_table, engagement_table):
    return _sc_call(student_id.astype(jnp.int32), engagement_id.astype(jnp.int32),
                    student_table, engagement_table)


# speedup vs baseline: 1.7200x; 1.3625x over previous
import functools

import jax
import jax.numpy as jnp
from jax import lax
from jax.experimental import pallas as pl
from jax.experimental.pallas import tpu as pltpu
from jax.experimental.pallas import tpu_sc as plsc

_NUM_BINS = 10000
_EMBED_DIM = 64
_BATCH = 16384

_info = plsc.get_sparse_core_info()
_NC, _NS, _L = _info.num_cores, _info.num_subcores, _info.num_lanes
_NW = _NC * _NS
_BPW = _BATCH // _NW
_CHUNK = 128
_KCH = _BPW // _CHUNK


def _sc_body(sid_hbm, eid_hbm, stab_hbm, etab_hbm, out_hbm,
             ids_v, sidx, eidx, comb, gsems, osem):
    wid = lax.axis_index("s") * _NC + lax.axis_index("c")
    base = wid * _BPW

    id0 = pltpu.async_copy(sid_hbm.at[pl.ds(base, _BPW)], ids_v.at[0], osem)
    id1 = pltpu.async_copy(eid_hbm.at[pl.ds(base, _BPW)], ids_v.at[1], osem)
    id0.wait()
    id1.wait()

    inv = 1.0 / _NUM_BINS

    def _mod(v):
        q = (v.astype(jnp.float32) * inv).astype(jnp.int32)
        r = v - q * _NUM_BINS
        r = jnp.where(r < 0, r + _NUM_BINS, r)
        return jnp.where(r >= _NUM_BINS, r - _NUM_BINS, r)

    gd = []
    for j in range(_KCH):
        for t in range(_CHUNK // _L):
            col = t * _L
            i0 = j * _CHUNK + col
            sidx[j, pl.ds(col, _L)] = _mod(ids_v[0, pl.ds(i0, _L)])
            eidx[j, pl.ds(col, _L)] = _mod(ids_v[1, pl.ds(i0, _L)])
        rows = pl.ds(j * _CHUNK, _CHUNK)
        gd.append(pltpu.async_copy(stab_hbm.at[sidx.at[j]], comb.at[0, rows], gsems.at[j]))
        gd.append(pltpu.async_copy(etab_hbm.at[eidx.at[j]], comb.at[1, rows], gsems.at[j]))
    od = []
    for j in range(_KCH):
        gd[2 * j].wait()
        gd[2 * j + 1].wait()
        rows = pl.ds(j * _CHUNK, _CHUNK)
        orows = pl.ds(base + j * _CHUNK, _CHUNK)
        od.append(pltpu.async_copy(comb.at[0, rows],
                                   out_hbm.at[orows, pl.ds(0, _EMBED_DIM)], osem))
        od.append(pltpu.async_copy(comb.at[1, rows],
                                   out_hbm.at[orows, pl.ds(_EMBED_DIM, _EMBED_DIM)], osem))
    for d in od:
        d.wait()


_sc_call = functools.partial(
    pl.kernel,
    mesh=plsc.VectorSubcoreMesh(core_axis_name="c", subcore_axis_name="s"),
    out_type=jax.ShapeDtypeStruct((_BATCH, 2 * _EMBED_DIM), jnp.float32),
    scratch_types=[
        pltpu.VMEM((2, _BPW), jnp.int32),
        pltpu.VMEM((_KCH, _CHUNK), jnp.int32),
        pltpu.VMEM((_KCH, _CHUNK), jnp.int32),
        pltpu.VMEM((2, _BPW, _EMBED_DIM), jnp.float32),
        pltpu.SemaphoreType.DMA((_KCH,)),
        pltpu.SemaphoreType.DMA,
    ],
    compiler_params=pltpu.CompilerParams(use_tc_tiling_on_sc=False),
)(_sc_body)


def kernel(student_id, engagement_id, student_table, engagement_table):
    return _sc_call(student_id.astype(jnp.int32), engagement_id.astype(jnp.int32),
                    student_table, engagement_table)
